# baseline (speedup 1.0000x reference)
import functools

import jax
import jax.numpy as jnp
from jax.experimental import pallas as pl
from jax.experimental.pallas import tpu as pltpu

B = 128
V = 100000
VC = 1024
NSTEPS = (V + VC - 1) // VC

import numpy as np

_NEG_INF = np.float32(-np.inf)
_TINY = np.float32(1.1754944e-38)
_BIG_I32 = np.int32(2147483647)


def _rotl(x, r):
    return jnp.bitwise_or(
        jnp.left_shift(x, jnp.uint32(r)), jnp.right_shift(x, jnp.uint32(32 - r))
    )


def _threefry_bits(n):
    ks0 = jnp.uint32(0)
    ks1 = jnp.uint32(1)
    ks2 = jnp.uint32(0x1BD11BDA ^ 0 ^ 1)
    x0 = jnp.zeros_like(n) + ks0
    x1 = n + ks1
    rots = ((13, 15, 26, 6), (17, 29, 16, 24))
    ks = (ks0, ks1, ks2)
    for i in range(5):
        for r in rots[i % 2]:
            x0 = x0 + x1
            x1 = _rotl(x1, r)
            x1 = jnp.bitwise_xor(x1, x0)
        x0 = x0 + ks[(i + 1) % 3]
        x1 = x1 + ks[(i + 2) % 3] + jnp.uint32(i + 1)
    return jnp.bitwise_xor(x0, x1)


def _gumbel_from_bits(bits):
    fb = jnp.bitwise_or(jnp.right_shift(bits, jnp.uint32(9)), jnp.uint32(0x3F800000))
    f = jax.lax.bitcast_convert_type(fb, jnp.float32) - jnp.float32(1.0)
    u = jnp.maximum(_TINY, f + _TINY)
    return -jnp.log(-jnp.log(u))


def _fused_kernel(
    logits_ref,
    actions_ref,
    lp_out,
    ent_out,
    mode_out,
    sample_out,
    m_acc,
    s_acc,
    t_acc,
    lp_acc,
    modev_acc,
    modei_acc,
    sampv_acc,
    sampi_acc,
):
    j = pl.program_id(0)

    x = logits_ref[...]
    col = jax.lax.broadcasted_iota(jnp.int32, (B, VC), 1) + j * VC
    valid = col < V
    xm = jnp.where(valid, x, _NEG_INF)

    cmax = jnp.max(xm, axis=1, keepdims=True)

    @pl.when(j == 0)
    def _init():
        m_acc[...] = jnp.full((B, 1), _NEG_INF, jnp.float32)
        s_acc[...] = jnp.zeros((B, 1), jnp.float32)
        t_acc[...] = jnp.zeros((B, 1), jnp.float32)
        lp_acc[...] = jnp.zeros((B, 1), jnp.float32)
        modev_acc[...] = jnp.full((B, 1), _NEG_INF, jnp.float32)
        modei_acc[...] = jnp.zeros((B, 1), jnp.int32)
        sampv_acc[...] = jnp.full((B, 1), _NEG_INF, jnp.float32)
        sampi_acc[...] = jnp.zeros((B, 1), jnp.int32)

    m_old = m_acc[...]
    m_new = jnp.maximum(m_old, cmax)
    scale = jnp.where(m_old == _NEG_INF, 0.0, jnp.exp(m_old - m_new))
    e = jnp.where(valid, jnp.exp(x - m_new), 0.0)
    ex = e * jnp.where(valid, x, 0.0)
    s_acc[...] = s_acc[...] * scale + jnp.sum(e, axis=1, keepdims=True)
    t_acc[...] = t_acc[...] * scale + jnp.sum(ex, axis=1, keepdims=True)
    m_acc[...] = m_new

    cidx = jnp.min(jnp.where(xm == cmax, col, _BIG_I32), axis=1, keepdims=True)
    better = cmax > modev_acc[...]
    modev_acc[...] = jnp.where(better, cmax, modev_acc[...])
    modei_acc[...] = jnp.where(better, cidx, modei_acc[...])

    a = actions_ref[...]
    hit = col == a
    lp_acc[...] = lp_acc[...] + jnp.sum(
        jnp.where(hit, x, 0.0), axis=1, keepdims=True
    )

    n = (jax.lax.broadcasted_iota(jnp.int32, (B, VC), 0) * V + col).astype(jnp.uint32)
    g = _gumbel_from_bits(_threefry_bits(n))
    y = jnp.where(valid, x + g, _NEG_INF)
    ymax = jnp.max(y, axis=1, keepdims=True)
    yidx = jnp.min(jnp.where(y == ymax, col, _BIG_I32), axis=1, keepdims=True)
    ybetter = ymax > sampv_acc[...]
    sampv_acc[...] = jnp.where(ybetter, ymax, sampv_acc[...])
    sampi_acc[...] = jnp.where(ybetter, yidx, sampi_acc[...])

    @pl.when(j == NSTEPS - 1)
    def _final():
        lse = m_acc[...] + jnp.log(s_acc[...])
        lp_out[...] = lp_acc[...] - lse
        ent_out[...] = lse - t_acc[...] / s_acc[...]
        mode_out[...] = modei_acc[...]
        sample_out[...] = sampi_acc[...]


@functools.partial(jax.jit)
def kernel(logits, actions):
    out_shapes = (
        jax.ShapeDtypeStruct((B, 1), jnp.float32),
        jax.ShapeDtypeStruct((B, 1), jnp.float32),
        jax.ShapeDtypeStruct((B, 1), jnp.int32),
        jax.ShapeDtypeStruct((B, 1), jnp.int32),
    )
    lp, ent, mode, sample = pl.pallas_call(
        _fused_kernel,
        grid=(NSTEPS,),
        in_specs=[
            pl.BlockSpec((B, VC), lambda j: (0, j)),
            pl.BlockSpec((B, 1), lambda j: (0, 0)),
        ],
        out_specs=[
            pl.BlockSpec((B, 1), lambda j: (0, 0)),
            pl.BlockSpec((B, 1), lambda j: (0, 0)),
            pl.BlockSpec((B, 1), lambda j: (0, 0)),
            pl.BlockSpec((B, 1), lambda j: (0, 0)),
        ],
        out_shape=out_shapes,
        scratch_shapes=[pltpu.VMEM((B, 1), jnp.float32)] * 5
        + [pltpu.VMEM((B, 1), jnp.int32)]
        + [pltpu.VMEM((B, 1), jnp.float32)]
        + [pltpu.VMEM((B, 1), jnp.int32)],
        compiler_params=pltpu.CompilerParams(
            dimension_semantics=("arbitrary",),
        ),
    )(logits, actions)
    return (lp, ent.reshape(B), mode, sample)
